# Initial kernel scaffold; baseline (speedup 1.0000x reference)
import jax
import jax.numpy as jnp
from jax.experimental import pallas as pl


def kernel(x, table):
    raise NotImplementedError("write your pallas kernel here")


# speedup vs baseline: 1.2828x; 1.2828x over previous
import functools

import jax
import jax.numpy as jnp
from jax import lax
from jax.experimental import pallas as pl
from jax.experimental.pallas import tpu as pltpu
from jax.experimental.pallas import tpu_sc as plsc
from jax._src.pallas import primitives as _pl_prims

_EPS = 1e-12


def _l2normalize(v):
    s = jnp.sum(v * v, axis=1, keepdims=True)
    n = jnp.maximum(jnp.sqrt(s), _EPS)
    return v * _pl_prims.reciprocal(n, approx=True)


def _prep_body(t_ref, ew_ref, ewt_ref, bias_ref):
    t = t_ref[...]
    e = _l2normalize(t)
    ew_ref[...] = e
    et = e.T
    ewt_ref[...] = et
    bias_ref[...] = jnp.sum(et * et, axis=0, keepdims=True)


def _prep(table):
    k, d = table.shape
    kb = 1024
    return pl.pallas_call(
        _prep_body,
        grid=(k // kb,),
        in_specs=[pl.BlockSpec((kb, d), lambda i: (i, 0))],
        out_specs=[
            pl.BlockSpec((kb, d), lambda i: (i, 0)),
            pl.BlockSpec((d, kb), lambda i: (0, i)),
            pl.BlockSpec((1, kb), lambda i: (0, i)),
        ],
        out_shape=[
            jax.ShapeDtypeStruct((k, d), jnp.float32),
            jax.ShapeDtypeStruct((d, k), jnp.float32),
            jax.ShapeDtypeStruct((1, k), jnp.float32),
        ],
    )(table)


_BN = 256
_BK = 2048


def _vq_body(x_ref, ewt_ref, bias_ref, idx_ref, loss_ref, *, bn, bk, k):
    i = pl.program_id(0)
    x = x_ref[...]
    xn = _l2normalize(x)
    xn2 = jnp.sum(xn * xn, axis=1, keepdims=True)

    run_mb = jnp.full((bn, 1), jnp.inf, jnp.float32)
    run_val = jnp.full((bn, 1), jnp.inf, jnp.float32)
    run_idx = jnp.zeros((bn, 1), jnp.int32)
    xb = xn.astype(jnp.bfloat16)
    for j in range(k // bk):
        et = ewt_ref[:, j * bk:(j + 1) * bk]
        sc = jnp.dot(xb, et.astype(jnp.bfloat16),
                     preferred_element_type=jnp.float32)
        d = (xn2 + bias_ref[:, j * bk:(j + 1) * bk]) - 2.0 * sc
        mb = jnp.min(d, axis=1, keepdims=True)
        ids = lax.broadcasted_iota(jnp.int32, (bn, bk), 1)
        eq = d == mb
        am = jnp.min(jnp.where(eq, ids, k), axis=1, keepdims=True)
        vs = mb
        upd = mb < run_mb
        run_mb = jnp.where(upd, mb, run_mb)
        run_val = jnp.where(upd, vs, run_val)
        run_idx = jnp.where(upd, am + j * bk, run_idx)

    idx_ref[...] = run_idx

    @pl.when(i == 0)
    def _init():
        loss_ref[...] = jnp.zeros_like(loss_ref)

    loss_ref[...] += jnp.sum(run_val).reshape(1, 1)


def _vq(x, ewt, bias):
    n, d = x.shape
    k = ewt.shape[1]
    body = functools.partial(_vq_body, bn=_BN, bk=_BK, k=k)
    return pl.pallas_call(
        body,
        grid=(n // _BN,),
        in_specs=[
            pl.BlockSpec((_BN, d), lambda i: (i, 0)),
            pl.BlockSpec((d, k), lambda i: (0, 0)),
            pl.BlockSpec((1, k), lambda i: (0, 0)),
        ],
        out_specs=[
            pl.BlockSpec((_BN, 1), lambda i: (i, 0)),
            pl.BlockSpec((1, 1), lambda i: (0, 0)),
        ],
        out_shape=[
            jax.ShapeDtypeStruct((n, 1), jnp.int32),
            jax.ShapeDtypeStruct((1, 1), jnp.float32),
        ],
        compiler_params=pltpu.CompilerParams(
            dimension_semantics=("arbitrary",)),
    )(x, ewt, bias)


_CHUNK = 128


def _gather_body(ew_hbm, idx_hbm, out_hbm, idx_v, rows_v, sem, *, per, chunk):
    wid = lax.axis_index("s") * 2 + lax.axis_index("c")
    base = wid * per
    pltpu.sync_copy(idx_hbm.at[pl.ds(base, per)], idx_v)

    def body(j, carry):
        src = ew_hbm.at[idx_v.at[pl.ds(j * chunk, chunk)]]
        pltpu.async_copy(src, rows_v, sem).wait()
        pltpu.sync_copy(rows_v, out_hbm.at[pl.ds(base + j * chunk, chunk)])
        return carry

    lax.fori_loop(0, per // chunk, body, 0)


def _gather(ew, idx):
    k, d = ew.shape
    n = idx.shape[0]
    nw = 32
    per = n // nw
    mesh = plsc.VectorSubcoreMesh(core_axis_name="c", subcore_axis_name="s")
    body = functools.partial(_gather_body, per=per, chunk=_CHUNK)
    fn = pl.kernel(
        body, mesh=mesh,
        out_type=jax.ShapeDtypeStruct((n, d), jnp.float32),
        scratch_types=[
            pltpu.VMEM((per,), jnp.int32),
            pltpu.VMEM((_CHUNK, d), jnp.float32),
            pltpu.SemaphoreType.DMA,
        ],
    )
    return fn(ew, idx)


def kernel(x, table):
    n, d = x.shape
    ew, ewt, bias = _prep(table)
    idx2d, loss_acc = _vq(x, ewt, bias)
    idx = idx2d.reshape(n)
    quantized = _gather(ew, idx)
    m = loss_acc[0, 0] / jnp.float32(n * d)
    loss = m + 0.25 * m
    return quantized, loss, idx
